# baseline (speedup 1.0000x reference)
import functools

import jax
import jax.numpy as jnp
from jax import lax
from jax.experimental import pallas as pl
from jax.experimental.pallas import tpu as pltpu
from jax.experimental.pallas import tpu_sc as plsc

N = 10000
E = 320000
G = 64
D = 128

NC = 2
NS = 16
NW = NC * NS
CHUNK = 128

EC = 80
E_PAD = NW * EC * CHUNK
SRC_PAD_ROWS = 1
N_PAD = 10112
RPT = N_PAD // NS

PC = 3
P_PAD = NW * PC * CHUNK
G_PAD = 128
G_RPT = G_PAD // NS


def _seg_sum_kernel(n_chunks, out_rows):
    mesh = plsc.VectorSubcoreMesh(core_axis_name="c", subcore_axis_name="s")
    rpt = out_rows // NS

    @functools.partial(
        pl.kernel,
        mesh=mesh,
        out_type=jax.ShapeDtypeStruct((NC, out_rows, D), jnp.float32),
        scratch_types=[
            pltpu.VMEM((n_chunks + SRC_PAD_ROWS, CHUNK), jnp.int32),
            pltpu.VMEM((n_chunks, CHUNK), jnp.int32),
            pltpu.VMEM((CHUNK, D), jnp.float32),
            pltpu.VMEM_SHARED((out_rows, D), jnp.float32),
            pltpu.SemaphoreType.DMA,
        ],
    )
    def k(table, src, dst, zeros, out, src_v, dst_v, rows_v, agg_sh, sem):
        c = lax.axis_index("c")
        s = lax.axis_index("s")
        wid = s * NC + c
        pltpu.sync_copy(zeros.at[pl.ds(0, rpt)], agg_sh.at[pl.ds(s * rpt, rpt)])
        pltpu.sync_copy(src.at[wid], src_v.at[pl.ds(0, n_chunks)])
        pltpu.sync_copy(dst.at[wid], dst_v)
        plsc.subcore_barrier()

        def body(j, carry):
            pltpu.async_copy(table.at[src_v.at[j]], rows_v, sem).wait()
            pltpu.sync_copy(rows_v, agg_sh.at[dst_v.at[j]], add=True)
            return carry

        lax.fori_loop(0, n_chunks, body, 0)
        plsc.subcore_barrier()
        pltpu.sync_copy(agg_sh.at[pl.ds(s * rpt, rpt)],
                        out.at[c, pl.ds(s * rpt, rpt)])

    return k


_edge_seg_sum = _seg_sum_kernel(EC, N_PAD)


def _pool_kernel():
    mesh = plsc.VectorSubcoreMesh(core_axis_name="c", subcore_axis_name="s")

    @functools.partial(
        pl.kernel,
        mesh=mesh,
        out_type=(
            jax.ShapeDtypeStruct((NC, G_PAD, D), jnp.float32),
            jax.ShapeDtypeStruct((NC, G_PAD, D), jnp.float32),
        ),
        scratch_types=[
            pltpu.VMEM((PC, CHUNK), jnp.int32),
            pltpu.VMEM((PC, CHUNK), jnp.int32),
            pltpu.VMEM((CHUNK, D), jnp.float32),
            pltpu.VMEM((CHUNK, D), jnp.float32),
            pltpu.VMEM_SHARED((G_PAD, D), jnp.float32),
            pltpu.VMEM_SHARED((G_PAD, D), jnp.float32),
            pltpu.SemaphoreType.DMA,
        ],
    )
    def k(table, src, dst, zeros, ones, sums_out, cnts_out,
          src_v, dst_v, rows_v, ones_v, sums_sh, cnts_sh, sem):
        c = lax.axis_index("c")
        s = lax.axis_index("s")
        wid = s * NC + c
        pltpu.sync_copy(zeros.at[pl.ds(0, G_RPT)],
                        sums_sh.at[pl.ds(s * G_RPT, G_RPT)])
        pltpu.sync_copy(zeros.at[pl.ds(0, G_RPT)],
                        cnts_sh.at[pl.ds(s * G_RPT, G_RPT)])
        pltpu.sync_copy(src.at[wid], src_v)
        pltpu.sync_copy(dst.at[wid], dst_v)
        pltpu.sync_copy(ones, ones_v)
        plsc.subcore_barrier()

        def body(j, carry):
            pltpu.async_copy(table.at[src_v.at[j]], rows_v, sem).wait()
            pltpu.sync_copy(rows_v, sums_sh.at[dst_v.at[j]], add=True)
            pltpu.sync_copy(ones_v, cnts_sh.at[dst_v.at[j]], add=True)
            return carry

        lax.fori_loop(0, PC, body, 0)
        plsc.subcore_barrier()
        pltpu.sync_copy(sums_sh.at[pl.ds(s * G_RPT, G_RPT)],
                        sums_out.at[c, pl.ds(s * G_RPT, G_RPT)])
        pltpu.sync_copy(cnts_sh.at[pl.ds(s * G_RPT, G_RPT)],
                        cnts_out.at[c, pl.ds(s * G_RPT, G_RPT)])

    return k


_pool_seg = _pool_kernel()

BN = 632
TC_GRID = N_PAD // BN


def _elu(v):
    return jnp.where(v > 0, v, jnp.exp(v) - 1.0)


def _dense_body(p_ref, h_ref, wrel_ref, wroot_ref, b_ref, out_ref):
    agg = p_ref[0] + p_ref[1]
    acc = jnp.dot(agg, wrel_ref[...], preferred_element_type=jnp.float32)
    acc += jnp.dot(h_ref[...], wroot_ref[...], preferred_element_type=jnp.float32)
    out_ref[...] = _elu(acc + b_ref[...])


def _dense_layer(p, h, wrel, wroot, b, d_out):
    d_in = h.shape[-1]
    return pl.pallas_call(
        _dense_body,
        grid=(TC_GRID,),
        in_specs=[
            pl.BlockSpec((2, BN, D), lambda i: (0, i, 0)),
            pl.BlockSpec((BN, d_in), lambda i: (i, 0)),
            pl.BlockSpec((D, d_out), lambda i: (0, 0)),
            pl.BlockSpec((d_in, d_out), lambda i: (0, 0)),
            pl.BlockSpec((1, d_out), lambda i: (0, 0)),
        ],
        out_specs=pl.BlockSpec((BN, d_out), lambda i: (i, 0)),
        out_shape=jax.ShapeDtypeStruct((N_PAD, d_out), jnp.float32),
    )(p, h, wrel, wroot, b)


def _dense2_body(p_ref, h_ref, wrel_ref, wroot_ref, b_ref, w3rel_ref,
                 h2_ref, y3_ref):
    agg = p_ref[0] + p_ref[1]
    acc = jnp.dot(agg, wrel_ref[...], preferred_element_type=jnp.float32)
    acc += jnp.dot(h_ref[...], wroot_ref[...], preferred_element_type=jnp.float32)
    h2 = _elu(acc + b_ref[...])
    h2_ref[...] = h2
    y3_ref[...] = jnp.dot(h2, w3rel_ref[...], preferred_element_type=jnp.float32)


def _dense_layer2(p, h, wrel, wroot, b, w3rel):
    return pl.pallas_call(
        _dense2_body,
        grid=(TC_GRID,),
        in_specs=[
            pl.BlockSpec((2, BN, D), lambda i: (0, i, 0)),
            pl.BlockSpec((BN, D), lambda i: (i, 0)),
            pl.BlockSpec((D, 2 * D), lambda i: (0, 0)),
            pl.BlockSpec((D, 2 * D), lambda i: (0, 0)),
            pl.BlockSpec((1, 2 * D), lambda i: (0, 0)),
            pl.BlockSpec((2 * D, D), lambda i: (0, 0)),
        ],
        out_specs=[
            pl.BlockSpec((BN, 2 * D), lambda i: (i, 0)),
            pl.BlockSpec((BN, D), lambda i: (i, 0)),
        ],
        out_shape=[
            jax.ShapeDtypeStruct((N_PAD, 2 * D), jnp.float32),
            jax.ShapeDtypeStruct((N_PAD, D), jnp.float32),
        ],
    )(p, h, wrel, wroot, b, w3rel)


def _dense3_body(p_ref, h_ref, wroot_ref, b_ref, out_ref):
    agg = p_ref[0] + p_ref[1]
    acc = agg + jnp.dot(h_ref[...], wroot_ref[...],
                        preferred_element_type=jnp.float32)
    out_ref[...] = _elu(acc + b_ref[...])


def _dense_layer3(p, h, wroot, b):
    return pl.pallas_call(
        _dense3_body,
        grid=(TC_GRID,),
        in_specs=[
            pl.BlockSpec((2, BN, D), lambda i: (0, i, 0)),
            pl.BlockSpec((BN, 2 * D), lambda i: (i, 0)),
            pl.BlockSpec((2 * D, D), lambda i: (0, 0)),
            pl.BlockSpec((1, D), lambda i: (0, 0)),
        ],
        out_specs=pl.BlockSpec((BN, D), lambda i: (i, 0)),
        out_shape=jax.ShapeDtypeStruct((N_PAD, D), jnp.float32),
    )(p, h, wroot, b)


def _finalize_body(s_ref, c_ref, out_ref):
    sums = s_ref[0] + s_ref[1]
    cnts = c_ref[0] + c_ref[1]
    out_ref[...] = sums / jnp.maximum(cnts, 1.0)


def _finalize(sums, cnts):
    return pl.pallas_call(
        _finalize_body,
        grid=(1,),
        in_specs=[
            pl.BlockSpec((2, G, D), lambda i: (0, 0, 0)),
            pl.BlockSpec((2, G, D), lambda i: (0, 0, 0)),
        ],
        out_specs=pl.BlockSpec((G, D), lambda i: (0, 0)),
        out_shape=jax.ShapeDtypeStruct((G, D), jnp.float32),
    )(sums, cnts)


def kernel(x, edge_index, batch, W1_rel, W1_root, b1, W2_rel, W2_root, b2,
           W3_rel, W3_root, b3):
    f32 = jnp.float32
    x_pad = jnp.concatenate([x, jnp.zeros((N_PAD - N, D), f32)], axis=0)
    src = edge_index[0]
    dst = edge_index[1]
    src_p = jnp.concatenate(
        [src, jnp.zeros((E_PAD - E,), jnp.int32)]).reshape(NW, EC, CHUNK)
    pad_dst = N + (jnp.arange(E_PAD - E, dtype=jnp.int32) % (N_PAD - N))
    dst_p = jnp.concatenate([dst, pad_dst]).reshape(NW, EC, CHUNK)
    zeros_hbm = jnp.zeros((RPT, D), f32)
    ones_hbm = jnp.ones((CHUNK, D), f32)
    node_idx = jnp.concatenate(
        [jnp.arange(N, dtype=jnp.int32),
         jnp.zeros((P_PAD - N,), jnp.int32)]).reshape(NW, PC, CHUNK)
    batch_p = jnp.concatenate(
        [batch, jnp.full((P_PAD - N,), G, jnp.int32)]).reshape(NW, PC, CHUNK)
    b1r = b1.reshape(1, -1)
    b2r = b2.reshape(1, -1)
    b3r = b3.reshape(1, -1)

    p1 = _edge_seg_sum(x_pad, src_p, dst_p, zeros_hbm)
    h1 = _dense_layer(p1, x_pad, W1_rel, W1_root, b1r, D)
    p2 = _edge_seg_sum(h1, src_p, dst_p, zeros_hbm)
    h2, y3 = _dense_layer2(p2, h1, W2_rel, W2_root, b2r, W3_rel)
    p3 = _edge_seg_sum(y3, src_p, dst_p, zeros_hbm)
    h3 = _dense_layer3(p3, h2, W3_root, b3r)
    sums, cnts = _pool_seg(h3, node_idx, batch_p, zeros_hbm, ones_hbm)
    return _finalize(sums, cnts)


# speedup vs baseline: 3.1314x; 3.1314x over previous
import functools

import jax
import jax.numpy as jnp
from jax import lax
from jax.experimental import pallas as pl
from jax.experimental.pallas import tpu as pltpu
from jax.experimental.pallas import tpu_sc as plsc

N = 10000
E = 320000
G = 64
D = 128

NC = 2
NS = 16
NW = NC * NS
CHUNK = 128

EC = 80
E_PAD = NW * EC * CHUNK
SRC_PAD_ROWS = 1
N_PAD = 10112
RPT = N_PAD // NS

PC = 3
P_PAD = NW * PC * CHUNK
G_PAD = 128
G_RPT = G_PAD // NS


def _seg_sum_kernel(n_chunks, out_rows):
    mesh = plsc.VectorSubcoreMesh(core_axis_name="c", subcore_axis_name="s")
    rpt = out_rows // NS

    @functools.partial(
        pl.kernel,
        mesh=mesh,
        out_type=jax.ShapeDtypeStruct((NC, out_rows, D), jnp.float32),
        scratch_types=[
            pltpu.VMEM((n_chunks + SRC_PAD_ROWS, CHUNK), jnp.int32),
            pltpu.VMEM((n_chunks, CHUNK), jnp.int32),
            pltpu.VMEM((CHUNK, D), jnp.float32),
            pltpu.VMEM_SHARED((out_rows, D), jnp.float32),
            pltpu.SemaphoreType.DMA,
        ],
    )
    def k(table, src, dst, zeros, out, src_v, dst_v, rows_v, agg_sh, sem):
        c = lax.axis_index("c")
        s = lax.axis_index("s")
        wid = s * NC + c
        pltpu.sync_copy(zeros.at[pl.ds(0, rpt)], agg_sh.at[pl.ds(s * rpt, rpt)])
        pltpu.sync_copy(src.at[wid], src_v.at[pl.ds(0, n_chunks)])
        pltpu.sync_copy(dst.at[wid], dst_v)
        plsc.subcore_barrier()

        def body(j, carry):
            pltpu.async_copy(table.at[src_v.at[j]], rows_v, sem).wait()
            pltpu.sync_copy(rows_v, agg_sh.at[dst_v.at[j]], add=True)
            return carry

        lax.fori_loop(0, n_chunks, body, 0)
        plsc.subcore_barrier()
        pltpu.sync_copy(agg_sh.at[pl.ds(s * rpt, rpt)],
                        out.at[c, pl.ds(s * rpt, rpt)])

    return k


_edge_seg_sum = _seg_sum_kernel(EC, N_PAD)


def _pool_kernel():
    mesh = plsc.VectorSubcoreMesh(core_axis_name="c", subcore_axis_name="s")

    @functools.partial(
        pl.kernel,
        mesh=mesh,
        out_type=(
            jax.ShapeDtypeStruct((NC, G_PAD, D), jnp.float32),
            jax.ShapeDtypeStruct((NC, G_PAD, D), jnp.float32),
        ),
        scratch_types=[
            pltpu.VMEM((PC, CHUNK), jnp.int32),
            pltpu.VMEM((PC, CHUNK), jnp.int32),
            pltpu.VMEM((CHUNK, D), jnp.float32),
            pltpu.VMEM((CHUNK, D), jnp.float32),
            pltpu.VMEM_SHARED((G_PAD, D), jnp.float32),
            pltpu.VMEM_SHARED((G_PAD, D), jnp.float32),
            pltpu.SemaphoreType.DMA,
        ],
    )
    def k(table, src, dst, zeros, ones, sums_out, cnts_out,
          src_v, dst_v, rows_v, ones_v, sums_sh, cnts_sh, sem):
        c = lax.axis_index("c")
        s = lax.axis_index("s")
        wid = s * NC + c
        pltpu.sync_copy(zeros.at[pl.ds(0, G_RPT)],
                        sums_sh.at[pl.ds(s * G_RPT, G_RPT)])
        pltpu.sync_copy(zeros.at[pl.ds(0, G_RPT)],
                        cnts_sh.at[pl.ds(s * G_RPT, G_RPT)])
        pltpu.sync_copy(src.at[wid], src_v)
        pltpu.sync_copy(dst.at[wid], dst_v)
        pltpu.sync_copy(ones, ones_v)
        plsc.subcore_barrier()

        def body(j, carry):
            pltpu.async_copy(table.at[src_v.at[j]], rows_v, sem).wait()
            pltpu.sync_copy(rows_v, sums_sh.at[dst_v.at[j]], add=True)
            pltpu.sync_copy(ones_v, cnts_sh.at[dst_v.at[j]], add=True)
            return carry

        lax.fori_loop(0, PC, body, 0)
        plsc.subcore_barrier()
        pltpu.sync_copy(sums_sh.at[pl.ds(s * G_RPT, G_RPT)],
                        sums_out.at[c, pl.ds(s * G_RPT, G_RPT)])
        pltpu.sync_copy(cnts_sh.at[pl.ds(s * G_RPT, G_RPT)],
                        cnts_out.at[c, pl.ds(s * G_RPT, G_RPT)])

    return k


_pool_seg = _pool_kernel()

BN = 632
TC_GRID = N_PAD // BN


def _elu(v):
    return jnp.where(v > 0, v, jnp.exp(v) - 1.0)


def _dense_body(p_ref, h_ref, wrel_ref, wroot_ref, b_ref, out_ref):
    agg = p_ref[0] + p_ref[1]
    acc = jnp.dot(agg, wrel_ref[...], preferred_element_type=jnp.float32)
    acc += jnp.dot(h_ref[...], wroot_ref[...], preferred_element_type=jnp.float32)
    out_ref[...] = _elu(acc + b_ref[...])


def _dense_layer(p, h, wrel, wroot, b, d_out):
    d_in = h.shape[-1]
    return pl.pallas_call(
        _dense_body,
        grid=(TC_GRID,),
        in_specs=[
            pl.BlockSpec((2, BN, D), lambda i: (0, i, 0)),
            pl.BlockSpec((BN, d_in), lambda i: (i, 0)),
            pl.BlockSpec((D, d_out), lambda i: (0, 0)),
            pl.BlockSpec((d_in, d_out), lambda i: (0, 0)),
            pl.BlockSpec((1, d_out), lambda i: (0, 0)),
        ],
        out_specs=pl.BlockSpec((BN, d_out), lambda i: (i, 0)),
        out_shape=jax.ShapeDtypeStruct((N_PAD, d_out), jnp.float32),
    )(p, h, wrel, wroot, b)


def _dense2_body(p_ref, h_ref, wrel_ref, wroot_ref, b_ref, w3rel_ref,
                 h2_ref, y3_ref):
    agg = p_ref[0] + p_ref[1]
    acc = jnp.dot(agg, wrel_ref[...], preferred_element_type=jnp.float32)
    acc += jnp.dot(h_ref[...], wroot_ref[...], preferred_element_type=jnp.float32)
    h2 = _elu(acc + b_ref[...])
    h2_ref[...] = h2
    y3_ref[...] = jnp.dot(h2, w3rel_ref[...], preferred_element_type=jnp.float32)


def _dense_layer2(p, h, wrel, wroot, b, w3rel):
    return pl.pallas_call(
        _dense2_body,
        grid=(TC_GRID,),
        in_specs=[
            pl.BlockSpec((2, BN, D), lambda i: (0, i, 0)),
            pl.BlockSpec((BN, D), lambda i: (i, 0)),
            pl.BlockSpec((D, 2 * D), lambda i: (0, 0)),
            pl.BlockSpec((D, 2 * D), lambda i: (0, 0)),
            pl.BlockSpec((1, 2 * D), lambda i: (0, 0)),
            pl.BlockSpec((2 * D, D), lambda i: (0, 0)),
        ],
        out_specs=[
            pl.BlockSpec((BN, 2 * D), lambda i: (i, 0)),
            pl.BlockSpec((BN, D), lambda i: (i, 0)),
        ],
        out_shape=[
            jax.ShapeDtypeStruct((N_PAD, 2 * D), jnp.float32),
            jax.ShapeDtypeStruct((N_PAD, D), jnp.float32),
        ],
    )(p, h, wrel, wroot, b, w3rel)


def _dense3_body(p_ref, h_ref, wroot_ref, b_ref, out_ref):
    agg = p_ref[0] + p_ref[1]
    acc = agg + jnp.dot(h_ref[...], wroot_ref[...],
                        preferred_element_type=jnp.float32)
    out_ref[...] = _elu(acc + b_ref[...])


def _dense_layer3(p, h, wroot, b):
    return pl.pallas_call(
        _dense3_body,
        grid=(TC_GRID,),
        in_specs=[
            pl.BlockSpec((2, BN, D), lambda i: (0, i, 0)),
            pl.BlockSpec((BN, 2 * D), lambda i: (i, 0)),
            pl.BlockSpec((2 * D, D), lambda i: (0, 0)),
            pl.BlockSpec((1, D), lambda i: (0, 0)),
        ],
        out_specs=pl.BlockSpec((BN, D), lambda i: (i, 0)),
        out_shape=jax.ShapeDtypeStruct((N_PAD, D), jnp.float32),
    )(p, h, wroot, b)


def _finalize_body(s_ref, c_ref, out_ref):
    sums = s_ref[0] + s_ref[1]
    cnts = c_ref[0] + c_ref[1]
    out_ref[...] = sums / jnp.maximum(cnts, 1.0)


def _finalize(sums, cnts):
    return pl.pallas_call(
        _finalize_body,
        grid=(1,),
        in_specs=[
            pl.BlockSpec((2, G, D), lambda i: (0, 0, 0)),
            pl.BlockSpec((2, G, D), lambda i: (0, 0, 0)),
        ],
        out_specs=pl.BlockSpec((G, D), lambda i: (0, 0)),
        out_shape=jax.ShapeDtypeStruct((G, D), jnp.float32),
    )(sums, cnts)


def kernel(x, edge_index, batch, W1_rel, W1_root, b1, W2_rel, W2_root, b2,
           W3_rel, W3_root, b3):
    f32 = jnp.float32
    x_pad = jnp.concatenate([x, jnp.zeros((N_PAD - N, D), f32)], axis=0)
    src = edge_index[0]
    dst = edge_index[1]
    pad_src = jnp.arange(E_PAD - E, dtype=jnp.int32) % N
    src_p = jnp.concatenate([src, pad_src]).reshape(NW, EC, CHUNK)
    pad_dst = N + (jnp.arange(E_PAD - E, dtype=jnp.int32) % (N_PAD - N))
    dst_p = jnp.concatenate([dst, pad_dst]).reshape(NW, EC, CHUNK)
    zeros_hbm = jnp.zeros((RPT, D), f32)
    ones_hbm = jnp.ones((CHUNK, D), f32)
    node_idx = jnp.concatenate(
        [jnp.arange(N, dtype=jnp.int32),
         jnp.arange(P_PAD - N, dtype=jnp.int32) % N]).reshape(NW, PC, CHUNK)
    batch_p = jnp.concatenate(
        [batch,
         G + jnp.arange(P_PAD - N, dtype=jnp.int32) % (G_PAD - G)]
    ).reshape(NW, PC, CHUNK)
    b1r = b1.reshape(1, -1)
    b2r = b2.reshape(1, -1)
    b3r = b3.reshape(1, -1)

    p1 = _edge_seg_sum(x_pad, src_p, dst_p, zeros_hbm)
    h1 = _dense_layer(p1, x_pad, W1_rel, W1_root, b1r, D)
    p2 = _edge_seg_sum(h1, src_p, dst_p, zeros_hbm)
    h2, y3 = _dense_layer2(p2, h1, W2_rel, W2_root, b2r, W3_rel)
    p3 = _edge_seg_sum(y3, src_p, dst_p, zeros_hbm)
    h3 = _dense_layer3(p3, h2, W3_root, b3r)
    sums, cnts = _pool_seg(h3, node_idx, batch_p, zeros_hbm, ones_hbm)
    return _finalize(sums, cnts)


# speedup vs baseline: 3.5150x; 1.1225x over previous
import functools

import jax
import jax.numpy as jnp
from jax import lax
from jax.experimental import pallas as pl
from jax.experimental.pallas import tpu as pltpu
from jax.experimental.pallas import tpu_sc as plsc

N = 10000
E = 320000
G = 64
D = 128

NC = 2
NS = 16
NW = NC * NS
CHUNK = 128

EC = 80
E_PAD = NW * EC * CHUNK
N_PAD = 10112
RPT = N_PAD // NS

PC = 3
P_PAD = NW * PC * CHUNK
G_PAD = 128
G_RPT = G_PAD // NS


def _seg_sum_kernel(n_chunks, out_rows):
    mesh = plsc.VectorSubcoreMesh(core_axis_name="c", subcore_axis_name="s")
    rpt = out_rows // NS

    npass = 2
    hp = n_chunks // npass
    assert hp % 2 == 0 and hp % 8 == 0

    @functools.partial(
        pl.kernel,
        mesh=mesh,
        out_type=jax.ShapeDtypeStruct((NC, out_rows, D), jnp.float32),
        scratch_types=[
            pltpu.VMEM((hp, CHUNK), jnp.int32),
            pltpu.VMEM((hp, CHUNK), jnp.int32),
            pltpu.VMEM((CHUNK, D), jnp.float32),
            pltpu.VMEM((CHUNK, D), jnp.float32),
            pltpu.VMEM_SHARED((out_rows, D), jnp.float32),
            pltpu.SemaphoreType.DMA,
            pltpu.SemaphoreType.DMA,
        ],
    )
    def k(table, src, dst, zeros, out, src_v, dst_v, b0, b1, agg_sh, t0, t1):
        c = lax.axis_index("c")
        s = lax.axis_index("s")
        wid = s * NC + c
        pltpu.sync_copy(zeros.at[pl.ds(0, rpt)], agg_sh.at[pl.ds(s * rpt, rpt)])
        plsc.subcore_barrier()

        for p in range(npass):
            pltpu.sync_copy(src.at[wid, pl.ds(p * hp, hp)], src_v)
            pltpu.sync_copy(dst.at[wid, pl.ds(p * hp, hp)], dst_v)

            def body(g, carry):
                j = 2 * g
                pltpu.sync_copy(table.at[src_v.at[j]], b0)
                cp0 = pltpu.async_copy(b0, agg_sh.at[dst_v.at[j]], t0,
                                       add=True)
                pltpu.sync_copy(table.at[src_v.at[j + 1]], b1)
                cp1 = pltpu.async_copy(b1, agg_sh.at[dst_v.at[j + 1]], t1,
                                       add=True)
                cp0.wait()
                cp1.wait()
                return carry

            lax.fori_loop(0, hp // 2, body, 0)

        plsc.subcore_barrier()
        pltpu.sync_copy(agg_sh.at[pl.ds(s * rpt, rpt)],
                        out.at[c, pl.ds(s * rpt, rpt)])

    return k


_edge_seg_sum = _seg_sum_kernel(EC, N_PAD)


def _pool_kernel():
    mesh = plsc.VectorSubcoreMesh(core_axis_name="c", subcore_axis_name="s")

    @functools.partial(
        pl.kernel,
        mesh=mesh,
        out_type=(
            jax.ShapeDtypeStruct((NC, G_PAD, D), jnp.float32),
            jax.ShapeDtypeStruct((NC, G_PAD, D), jnp.float32),
        ),
        scratch_types=[
            pltpu.VMEM((PC, CHUNK), jnp.int32),
            pltpu.VMEM((PC, CHUNK), jnp.int32),
            pltpu.VMEM((CHUNK, D), jnp.float32),
            pltpu.VMEM((CHUNK, D), jnp.float32),
            pltpu.VMEM_SHARED((G_PAD, D), jnp.float32),
            pltpu.VMEM_SHARED((G_PAD, D), jnp.float32),
            pltpu.SemaphoreType.DMA,
        ],
    )
    def k(table, src, dst, zeros, ones, sums_out, cnts_out,
          src_v, dst_v, rows_v, ones_v, sums_sh, cnts_sh, sem):
        c = lax.axis_index("c")
        s = lax.axis_index("s")
        wid = s * NC + c
        pltpu.sync_copy(zeros.at[pl.ds(0, G_RPT)],
                        sums_sh.at[pl.ds(s * G_RPT, G_RPT)])
        pltpu.sync_copy(zeros.at[pl.ds(0, G_RPT)],
                        cnts_sh.at[pl.ds(s * G_RPT, G_RPT)])
        pltpu.sync_copy(src.at[wid], src_v)
        pltpu.sync_copy(dst.at[wid], dst_v)
        pltpu.sync_copy(ones, ones_v)
        plsc.subcore_barrier()

        def body(j, carry):
            pltpu.async_copy(table.at[src_v.at[j]], rows_v, sem).wait()
            pltpu.sync_copy(rows_v, sums_sh.at[dst_v.at[j]], add=True)
            pltpu.sync_copy(ones_v, cnts_sh.at[dst_v.at[j]], add=True)
            return carry

        lax.fori_loop(0, PC, body, 0)
        plsc.subcore_barrier()
        pltpu.sync_copy(sums_sh.at[pl.ds(s * G_RPT, G_RPT)],
                        sums_out.at[c, pl.ds(s * G_RPT, G_RPT)])
        pltpu.sync_copy(cnts_sh.at[pl.ds(s * G_RPT, G_RPT)],
                        cnts_out.at[c, pl.ds(s * G_RPT, G_RPT)])

    return k


_pool_seg = _pool_kernel()

BN = 632
TC_GRID = N_PAD // BN


def _elu(v):
    return jnp.where(v > 0, v, jnp.exp(v) - 1.0)


def _dense_body(p_ref, h_ref, wrel_ref, wroot_ref, b_ref, out_ref):
    agg = p_ref[0] + p_ref[1]
    acc = jnp.dot(agg, wrel_ref[...], preferred_element_type=jnp.float32)
    acc += jnp.dot(h_ref[...], wroot_ref[...], preferred_element_type=jnp.float32)
    out_ref[...] = _elu(acc + b_ref[...])


def _dense_layer(p, h, wrel, wroot, b, d_out):
    d_in = h.shape[-1]
    return pl.pallas_call(
        _dense_body,
        grid=(TC_GRID,),
        in_specs=[
            pl.BlockSpec((2, BN, D), lambda i: (0, i, 0)),
            pl.BlockSpec((BN, d_in), lambda i: (i, 0)),
            pl.BlockSpec((D, d_out), lambda i: (0, 0)),
            pl.BlockSpec((d_in, d_out), lambda i: (0, 0)),
            pl.BlockSpec((1, d_out), lambda i: (0, 0)),
        ],
        out_specs=pl.BlockSpec((BN, d_out), lambda i: (i, 0)),
        out_shape=jax.ShapeDtypeStruct((N_PAD, d_out), jnp.float32),
    )(p, h, wrel, wroot, b)


def _dense2_body(p_ref, h_ref, wrel_ref, wroot_ref, b_ref, w3rel_ref,
                 h2_ref, y3_ref):
    agg = p_ref[0] + p_ref[1]
    acc = jnp.dot(agg, wrel_ref[...], preferred_element_type=jnp.float32)
    acc += jnp.dot(h_ref[...], wroot_ref[...], preferred_element_type=jnp.float32)
    h2 = _elu(acc + b_ref[...])
    h2_ref[...] = h2
    y3_ref[...] = jnp.dot(h2, w3rel_ref[...], preferred_element_type=jnp.float32)


def _dense_layer2(p, h, wrel, wroot, b, w3rel):
    return pl.pallas_call(
        _dense2_body,
        grid=(TC_GRID,),
        in_specs=[
            pl.BlockSpec((2, BN, D), lambda i: (0, i, 0)),
            pl.BlockSpec((BN, D), lambda i: (i, 0)),
            pl.BlockSpec((D, 2 * D), lambda i: (0, 0)),
            pl.BlockSpec((D, 2 * D), lambda i: (0, 0)),
            pl.BlockSpec((1, 2 * D), lambda i: (0, 0)),
            pl.BlockSpec((2 * D, D), lambda i: (0, 0)),
        ],
        out_specs=[
            pl.BlockSpec((BN, 2 * D), lambda i: (i, 0)),
            pl.BlockSpec((BN, D), lambda i: (i, 0)),
        ],
        out_shape=[
            jax.ShapeDtypeStruct((N_PAD, 2 * D), jnp.float32),
            jax.ShapeDtypeStruct((N_PAD, D), jnp.float32),
        ],
    )(p, h, wrel, wroot, b, w3rel)


def _dense3_body(p_ref, h_ref, wroot_ref, b_ref, out_ref):
    agg = p_ref[0] + p_ref[1]
    acc = agg + jnp.dot(h_ref[...], wroot_ref[...],
                        preferred_element_type=jnp.float32)
    out_ref[...] = _elu(acc + b_ref[...])


def _dense_layer3(p, h, wroot, b):
    return pl.pallas_call(
        _dense3_body,
        grid=(TC_GRID,),
        in_specs=[
            pl.BlockSpec((2, BN, D), lambda i: (0, i, 0)),
            pl.BlockSpec((BN, 2 * D), lambda i: (i, 0)),
            pl.BlockSpec((2 * D, D), lambda i: (0, 0)),
            pl.BlockSpec((1, D), lambda i: (0, 0)),
        ],
        out_specs=pl.BlockSpec((BN, D), lambda i: (i, 0)),
        out_shape=jax.ShapeDtypeStruct((N_PAD, D), jnp.float32),
    )(p, h, wroot, b)


def _finalize_body(s_ref, c_ref, out_ref):
    sums = s_ref[0] + s_ref[1]
    cnts = c_ref[0] + c_ref[1]
    out_ref[...] = sums / jnp.maximum(cnts, 1.0)


def _finalize(sums, cnts):
    return pl.pallas_call(
        _finalize_body,
        grid=(1,),
        in_specs=[
            pl.BlockSpec((2, G, D), lambda i: (0, 0, 0)),
            pl.BlockSpec((2, G, D), lambda i: (0, 0, 0)),
        ],
        out_specs=pl.BlockSpec((G, D), lambda i: (0, 0)),
        out_shape=jax.ShapeDtypeStruct((G, D), jnp.float32),
    )(sums, cnts)


def kernel(x, edge_index, batch, W1_rel, W1_root, b1, W2_rel, W2_root, b2,
           W3_rel, W3_root, b3):
    f32 = jnp.float32
    x_pad = jnp.concatenate([x, jnp.zeros((N_PAD - N, D), f32)], axis=0)
    src = edge_index[0]
    dst = edge_index[1]
    pad_src = jnp.arange(E_PAD - E, dtype=jnp.int32) % N
    src_p = jnp.concatenate([src, pad_src]).reshape(NW, EC, CHUNK)
    pad_dst = N + (jnp.arange(E_PAD - E, dtype=jnp.int32) % (N_PAD - N))
    dst_p = jnp.concatenate([dst, pad_dst]).reshape(NW, EC, CHUNK)
    zeros_hbm = jnp.zeros((RPT, D), f32)
    ones_hbm = jnp.ones((CHUNK, D), f32)
    node_idx = jnp.concatenate(
        [jnp.arange(N, dtype=jnp.int32),
         jnp.arange(P_PAD - N, dtype=jnp.int32) % N]).reshape(NW, PC, CHUNK)
    batch_p = jnp.concatenate(
        [batch,
         G + jnp.arange(P_PAD - N, dtype=jnp.int32) % (G_PAD - G)]
    ).reshape(NW, PC, CHUNK)
    b1r = b1.reshape(1, -1)
    b2r = b2.reshape(1, -1)
    b3r = b3.reshape(1, -1)

    p1 = _edge_seg_sum(x_pad, src_p, dst_p, zeros_hbm)
    h1 = _dense_layer(p1, x_pad, W1_rel, W1_root, b1r, D)
    p2 = _edge_seg_sum(h1, src_p, dst_p, zeros_hbm)
    h2, y3 = _dense_layer2(p2, h1, W2_rel, W2_root, b2r, W3_rel)
    p3 = _edge_seg_sum(y3, src_p, dst_p, zeros_hbm)
    h3 = _dense_layer3(p3, h2, W3_root, b3r)
    sums, cnts = _pool_seg(h3, node_idx, batch_p, zeros_hbm, ones_hbm)
    return _finalize(sums, cnts)


# speedup vs baseline: 3.5943x; 1.0226x over previous
import functools

import jax
import jax.numpy as jnp
from jax import lax
from jax.experimental import pallas as pl
from jax.experimental.pallas import tpu as pltpu
from jax.experimental.pallas import tpu_sc as plsc

N = 10000
E = 320000
G = 64
D = 128

NC = 2
NS = 16
NW = NC * NS
CHUNK = 128

EC = 80
E_PAD = NW * EC * CHUNK
N_PAD = 10112
RPT = N_PAD // NS

PC = 3
P_PAD = NW * PC * CHUNK
G_PAD = 128
G_RPT = G_PAD // NS


def _seg_sum_kernel(n_chunks, out_rows):
    mesh = plsc.VectorSubcoreMesh(core_axis_name="c", subcore_axis_name="s")
    rpt = out_rows // NS

    npass = 2
    hp = n_chunks // npass
    assert hp % 2 == 0 and hp % 8 == 0

    @functools.partial(
        pl.kernel,
        mesh=mesh,
        out_type=jax.ShapeDtypeStruct((NC, out_rows, D), jnp.float32),
        scratch_types=[
            pltpu.VMEM((hp, CHUNK), jnp.int32),
            pltpu.VMEM((hp, CHUNK), jnp.int32),
            pltpu.VMEM((CHUNK, D), jnp.float32),
            pltpu.VMEM((CHUNK, D), jnp.float32),
            pltpu.VMEM_SHARED((out_rows, D), jnp.float32),
            pltpu.SemaphoreType.DMA,
            pltpu.SemaphoreType.DMA,
            pltpu.SemaphoreType.DMA,
            pltpu.SemaphoreType.DMA,
        ],
    )
    def k(table, src, dst, zeros, out, src_v, dst_v, b0, b1, agg_sh,
          t0, t1, g0, g1):
        c = lax.axis_index("c")
        s = lax.axis_index("s")
        wid = s * NC + c
        pltpu.sync_copy(zeros.at[pl.ds(0, rpt)], agg_sh.at[pl.ds(s * rpt, rpt)])
        plsc.subcore_barrier()

        for p in range(npass):
            pltpu.sync_copy(src.at[wid, pl.ds(p * hp, hp)], src_v)
            pltpu.sync_copy(dst.at[wid, pl.ds(p * hp, hp)], dst_v)

            def body(g, carry):
                j = 2 * g
                cg0 = pltpu.async_copy(table.at[src_v.at[j]], b0, g0)
                cg1 = pltpu.async_copy(table.at[src_v.at[j + 1]], b1, g1)
                cg0.wait()
                cs0 = pltpu.async_copy(b0, agg_sh.at[dst_v.at[j]], t0,
                                       add=True)
                cg1.wait()
                cs1 = pltpu.async_copy(b1, agg_sh.at[dst_v.at[j + 1]], t1,
                                       add=True)
                cs0.wait()
                cs1.wait()
                return carry

            lax.fori_loop(0, hp // 2, body, 0)

        plsc.subcore_barrier()
        pltpu.sync_copy(agg_sh.at[pl.ds(s * rpt, rpt)],
                        out.at[c, pl.ds(s * rpt, rpt)])

    return k


_edge_seg_sum = _seg_sum_kernel(EC, N_PAD)


def _pool_kernel():
    mesh = plsc.VectorSubcoreMesh(core_axis_name="c", subcore_axis_name="s")

    @functools.partial(
        pl.kernel,
        mesh=mesh,
        out_type=(
            jax.ShapeDtypeStruct((NC, G_PAD, D), jnp.float32),
            jax.ShapeDtypeStruct((NC, G_PAD, D), jnp.float32),
        ),
        scratch_types=[
            pltpu.VMEM((PC, CHUNK), jnp.int32),
            pltpu.VMEM((PC, CHUNK), jnp.int32),
            pltpu.VMEM((CHUNK, D), jnp.float32),
            pltpu.VMEM((CHUNK, D), jnp.float32),
            pltpu.VMEM_SHARED((G_PAD, D), jnp.float32),
            pltpu.VMEM_SHARED((G_PAD, D), jnp.float32),
            pltpu.SemaphoreType.DMA,
        ],
    )
    def k(table, src, dst, zeros, ones, sums_out, cnts_out,
          src_v, dst_v, rows_v, ones_v, sums_sh, cnts_sh, sem):
        c = lax.axis_index("c")
        s = lax.axis_index("s")
        wid = s * NC + c
        pltpu.sync_copy(zeros.at[pl.ds(0, G_RPT)],
                        sums_sh.at[pl.ds(s * G_RPT, G_RPT)])
        pltpu.sync_copy(zeros.at[pl.ds(0, G_RPT)],
                        cnts_sh.at[pl.ds(s * G_RPT, G_RPT)])
        pltpu.sync_copy(src.at[wid], src_v)
        pltpu.sync_copy(dst.at[wid], dst_v)
        pltpu.sync_copy(ones, ones_v)
        plsc.subcore_barrier()

        def body(j, carry):
            pltpu.async_copy(table.at[src_v.at[j]], rows_v, sem).wait()
            pltpu.sync_copy(rows_v, sums_sh.at[dst_v.at[j]], add=True)
            pltpu.sync_copy(ones_v, cnts_sh.at[dst_v.at[j]], add=True)
            return carry

        lax.fori_loop(0, PC, body, 0)
        plsc.subcore_barrier()
        pltpu.sync_copy(sums_sh.at[pl.ds(s * G_RPT, G_RPT)],
                        sums_out.at[c, pl.ds(s * G_RPT, G_RPT)])
        pltpu.sync_copy(cnts_sh.at[pl.ds(s * G_RPT, G_RPT)],
                        cnts_out.at[c, pl.ds(s * G_RPT, G_RPT)])

    return k


_pool_seg = _pool_kernel()

BN = 632
TC_GRID = N_PAD // BN


def _elu(v):
    return jnp.where(v > 0, v, jnp.exp(v) - 1.0)


def _dense_body(p_ref, h_ref, wrel_ref, wroot_ref, b_ref, out_ref):
    agg = p_ref[0] + p_ref[1]
    acc = jnp.dot(agg, wrel_ref[...], preferred_element_type=jnp.float32)
    acc += jnp.dot(h_ref[...], wroot_ref[...], preferred_element_type=jnp.float32)
    out_ref[...] = _elu(acc + b_ref[...])


def _dense_layer(p, h, wrel, wroot, b, d_out):
    d_in = h.shape[-1]
    return pl.pallas_call(
        _dense_body,
        grid=(TC_GRID,),
        in_specs=[
            pl.BlockSpec((2, BN, D), lambda i: (0, i, 0)),
            pl.BlockSpec((BN, d_in), lambda i: (i, 0)),
            pl.BlockSpec((D, d_out), lambda i: (0, 0)),
            pl.BlockSpec((d_in, d_out), lambda i: (0, 0)),
            pl.BlockSpec((1, d_out), lambda i: (0, 0)),
        ],
        out_specs=pl.BlockSpec((BN, d_out), lambda i: (i, 0)),
        out_shape=jax.ShapeDtypeStruct((N_PAD, d_out), jnp.float32),
    )(p, h, wrel, wroot, b)


def _dense2_body(p_ref, h_ref, wrel_ref, wroot_ref, b_ref, w3rel_ref,
                 h2_ref, y3_ref):
    agg = p_ref[0] + p_ref[1]
    acc = jnp.dot(agg, wrel_ref[...], preferred_element_type=jnp.float32)
    acc += jnp.dot(h_ref[...], wroot_ref[...], preferred_element_type=jnp.float32)
    h2 = _elu(acc + b_ref[...])
    h2_ref[...] = h2
    y3_ref[...] = jnp.dot(h2, w3rel_ref[...], preferred_element_type=jnp.float32)


def _dense_layer2(p, h, wrel, wroot, b, w3rel):
    return pl.pallas_call(
        _dense2_body,
        grid=(TC_GRID,),
        in_specs=[
            pl.BlockSpec((2, BN, D), lambda i: (0, i, 0)),
            pl.BlockSpec((BN, D), lambda i: (i, 0)),
            pl.BlockSpec((D, 2 * D), lambda i: (0, 0)),
            pl.BlockSpec((D, 2 * D), lambda i: (0, 0)),
            pl.BlockSpec((1, 2 * D), lambda i: (0, 0)),
            pl.BlockSpec((2 * D, D), lambda i: (0, 0)),
        ],
        out_specs=[
            pl.BlockSpec((BN, 2 * D), lambda i: (i, 0)),
            pl.BlockSpec((BN, D), lambda i: (i, 0)),
        ],
        out_shape=[
            jax.ShapeDtypeStruct((N_PAD, 2 * D), jnp.float32),
            jax.ShapeDtypeStruct((N_PAD, D), jnp.float32),
        ],
    )(p, h, wrel, wroot, b, w3rel)


def _dense3_body(p_ref, h_ref, wroot_ref, b_ref, out_ref):
    agg = p_ref[0] + p_ref[1]
    acc = agg + jnp.dot(h_ref[...], wroot_ref[...],
                        preferred_element_type=jnp.float32)
    out_ref[...] = _elu(acc + b_ref[...])


def _dense_layer3(p, h, wroot, b):
    return pl.pallas_call(
        _dense3_body,
        grid=(TC_GRID,),
        in_specs=[
            pl.BlockSpec((2, BN, D), lambda i: (0, i, 0)),
            pl.BlockSpec((BN, 2 * D), lambda i: (i, 0)),
            pl.BlockSpec((2 * D, D), lambda i: (0, 0)),
            pl.BlockSpec((1, D), lambda i: (0, 0)),
        ],
        out_specs=pl.BlockSpec((BN, D), lambda i: (i, 0)),
        out_shape=jax.ShapeDtypeStruct((N_PAD, D), jnp.float32),
    )(p, h, wroot, b)


def _finalize_body(s_ref, c_ref, out_ref):
    sums = s_ref[0] + s_ref[1]
    cnts = c_ref[0] + c_ref[1]
    out_ref[...] = sums / jnp.maximum(cnts, 1.0)


def _finalize(sums, cnts):
    return pl.pallas_call(
        _finalize_body,
        grid=(1,),
        in_specs=[
            pl.BlockSpec((2, G, D), lambda i: (0, 0, 0)),
            pl.BlockSpec((2, G, D), lambda i: (0, 0, 0)),
        ],
        out_specs=pl.BlockSpec((G, D), lambda i: (0, 0)),
        out_shape=jax.ShapeDtypeStruct((G, D), jnp.float32),
    )(sums, cnts)


def kernel(x, edge_index, batch, W1_rel, W1_root, b1, W2_rel, W2_root, b2,
           W3_rel, W3_root, b3):
    f32 = jnp.float32
    x_pad = jnp.concatenate([x, jnp.zeros((N_PAD - N, D), f32)], axis=0)
    src = edge_index[0]
    dst = edge_index[1]
    pad_src = jnp.arange(E_PAD - E, dtype=jnp.int32) % N
    src_p = jnp.concatenate([src, pad_src]).reshape(NW, EC, CHUNK)
    pad_dst = N + (jnp.arange(E_PAD - E, dtype=jnp.int32) % (N_PAD - N))
    dst_p = jnp.concatenate([dst, pad_dst]).reshape(NW, EC, CHUNK)
    zeros_hbm = jnp.zeros((RPT, D), f32)
    ones_hbm = jnp.ones((CHUNK, D), f32)
    node_idx = jnp.concatenate(
        [jnp.arange(N, dtype=jnp.int32),
         jnp.arange(P_PAD - N, dtype=jnp.int32) % N]).reshape(NW, PC, CHUNK)
    batch_p = jnp.concatenate(
        [batch,
         G + jnp.arange(P_PAD - N, dtype=jnp.int32) % (G_PAD - G)]
    ).reshape(NW, PC, CHUNK)
    b1r = b1.reshape(1, -1)
    b2r = b2.reshape(1, -1)
    b3r = b3.reshape(1, -1)

    p1 = _edge_seg_sum(x_pad, src_p, dst_p, zeros_hbm)
    h1 = _dense_layer(p1, x_pad, W1_rel, W1_root, b1r, D)
    p2 = _edge_seg_sum(h1, src_p, dst_p, zeros_hbm)
    h2, y3 = _dense_layer2(p2, h1, W2_rel, W2_root, b2r, W3_rel)
    p3 = _edge_seg_sum(y3, src_p, dst_p, zeros_hbm)
    h3 = _dense_layer3(p3, h2, W3_root, b3r)
    sums, cnts = _pool_seg(h3, node_idx, batch_p, zeros_hbm, ones_hbm)
    return _finalize(sums, cnts)


# speedup vs baseline: 3.7037x; 1.0305x over previous
import functools

import jax
import jax.numpy as jnp
from jax import lax
from jax.experimental import pallas as pl
from jax.experimental.pallas import tpu as pltpu
from jax.experimental.pallas import tpu_sc as plsc

N = 10000
E = 320000
G = 64
D = 128

NC = 2
NS = 16
NW = NC * NS
CHUNK = 128

EC = 80
E_PAD = NW * EC * CHUNK
N_PAD = 10112
RPT = N_PAD // NS

def _seg_sum_kernel(n_chunks, out_rows):
    mesh = plsc.VectorSubcoreMesh(core_axis_name="c", subcore_axis_name="s")
    rpt = out_rows // NS

    npass = 2
    hp = n_chunks // npass
    assert hp % 2 == 0 and hp % 8 == 0

    @functools.partial(
        pl.kernel,
        mesh=mesh,
        out_type=jax.ShapeDtypeStruct((NC, out_rows, D), jnp.float32),
        scratch_types=[
            pltpu.VMEM((hp, CHUNK), jnp.int32),
            pltpu.VMEM((hp, CHUNK), jnp.int32),
            pltpu.VMEM((CHUNK, D), jnp.float32),
            pltpu.VMEM((CHUNK, D), jnp.float32),
            pltpu.VMEM_SHARED((out_rows, D), jnp.float32),
            pltpu.SemaphoreType.DMA,
            pltpu.SemaphoreType.DMA,
            pltpu.SemaphoreType.DMA,
            pltpu.SemaphoreType.DMA,
        ],
    )
    def k(table, src, dst, zeros, out, src_v, dst_v, b0, b1, agg_sh,
          t0, t1, g0, g1):
        c = lax.axis_index("c")
        s = lax.axis_index("s")
        wid = s * NC + c
        pltpu.sync_copy(zeros.at[pl.ds(0, rpt)], agg_sh.at[pl.ds(s * rpt, rpt)])
        plsc.subcore_barrier()

        for p in range(npass):
            pltpu.sync_copy(src.at[wid, pl.ds(p * hp, hp)], src_v)
            pltpu.sync_copy(dst.at[wid, pl.ds(p * hp, hp)], dst_v)

            def body(g, carry):
                j = 2 * g
                cg0 = pltpu.async_copy(table.at[src_v.at[j]], b0, g0)
                cg1 = pltpu.async_copy(table.at[src_v.at[j + 1]], b1, g1)
                cg0.wait()
                cs0 = pltpu.async_copy(b0, agg_sh.at[dst_v.at[j]], t0,
                                       add=True)
                cg1.wait()
                cs1 = pltpu.async_copy(b1, agg_sh.at[dst_v.at[j + 1]], t1,
                                       add=True)
                cs0.wait()
                cs1.wait()
                return carry

            lax.fori_loop(0, hp // 2, body, 0)

        plsc.subcore_barrier()
        pltpu.sync_copy(agg_sh.at[pl.ds(s * rpt, rpt)],
                        out.at[c, pl.ds(s * rpt, rpt)])

    return k


_edge_seg_sum = _seg_sum_kernel(EC, N_PAD)


BN = 632
TC_GRID = N_PAD // BN


def _elu(v):
    return jnp.where(v > 0, v, jnp.exp(v) - 1.0)


def _dense_body(p_ref, h_ref, wrel_ref, wroot_ref, b_ref, out_ref):
    agg = p_ref[0] + p_ref[1]
    acc = jnp.dot(agg, wrel_ref[...], preferred_element_type=jnp.float32)
    acc += jnp.dot(h_ref[...], wroot_ref[...], preferred_element_type=jnp.float32)
    out_ref[...] = _elu(acc + b_ref[...])


def _dense_layer(p, h, wrel, wroot, b, d_out):
    d_in = h.shape[-1]
    return pl.pallas_call(
        _dense_body,
        grid=(TC_GRID,),
        in_specs=[
            pl.BlockSpec((2, BN, D), lambda i: (0, i, 0)),
            pl.BlockSpec((BN, d_in), lambda i: (i, 0)),
            pl.BlockSpec((D, d_out), lambda i: (0, 0)),
            pl.BlockSpec((d_in, d_out), lambda i: (0, 0)),
            pl.BlockSpec((1, d_out), lambda i: (0, 0)),
        ],
        out_specs=pl.BlockSpec((BN, d_out), lambda i: (i, 0)),
        out_shape=jax.ShapeDtypeStruct((N_PAD, d_out), jnp.float32),
    )(p, h, wrel, wroot, b)


def _dense2_body(p_ref, h_ref, wrel_ref, wroot_ref, b_ref, w3rel_ref,
                 h2_ref, y3_ref):
    agg = p_ref[0] + p_ref[1]
    acc = jnp.dot(agg, wrel_ref[...], preferred_element_type=jnp.float32)
    acc += jnp.dot(h_ref[...], wroot_ref[...], preferred_element_type=jnp.float32)
    h2 = _elu(acc + b_ref[...])
    h2_ref[...] = h2
    y3_ref[...] = jnp.dot(h2, w3rel_ref[...], preferred_element_type=jnp.float32)


def _dense_layer2(p, h, wrel, wroot, b, w3rel):
    return pl.pallas_call(
        _dense2_body,
        grid=(TC_GRID,),
        in_specs=[
            pl.BlockSpec((2, BN, D), lambda i: (0, i, 0)),
            pl.BlockSpec((BN, D), lambda i: (i, 0)),
            pl.BlockSpec((D, 2 * D), lambda i: (0, 0)),
            pl.BlockSpec((D, 2 * D), lambda i: (0, 0)),
            pl.BlockSpec((1, 2 * D), lambda i: (0, 0)),
            pl.BlockSpec((2 * D, D), lambda i: (0, 0)),
        ],
        out_specs=[
            pl.BlockSpec((BN, 2 * D), lambda i: (i, 0)),
            pl.BlockSpec((BN, D), lambda i: (i, 0)),
        ],
        out_shape=[
            jax.ShapeDtypeStruct((N_PAD, 2 * D), jnp.float32),
            jax.ShapeDtypeStruct((N_PAD, D), jnp.float32),
        ],
    )(p, h, wrel, wroot, b, w3rel)


def _dense3_body(p_ref, h_ref, wroot_ref, b_ref, batch_ref, out_ref,
                 acc_ref, cnt_ref):
    i = pl.program_id(0)
    agg = p_ref[0] + p_ref[1]
    acc = agg + jnp.dot(h_ref[...], wroot_ref[...],
                        preferred_element_type=jnp.float32)
    h3 = _elu(acc + b_ref[...])
    gids = jax.lax.broadcasted_iota(jnp.int32, (BN, G), 1).astype(jnp.float32)
    m = (batch_ref[...] == gids).astype(jnp.float32)
    psum = jax.lax.dot_general(m, h3, (((0,), (0,)), ((), ())),
                               preferred_element_type=jnp.float32)
    pcnt = jnp.sum(m, axis=0)[:, None]

    @pl.when(i == 0)
    def _():
        acc_ref[...] = psum
        cnt_ref[...] = pcnt

    @pl.when(i > 0)
    def _():
        acc_ref[...] += psum
        cnt_ref[...] += pcnt

    @pl.when(i == TC_GRID - 1)
    def _():
        out_ref[...] = acc_ref[...] / jnp.maximum(cnt_ref[...], 1.0)


def _dense_layer3_pool(p, h, wroot, b, batch2d):
    return pl.pallas_call(
        _dense3_body,
        grid=(TC_GRID,),
        in_specs=[
            pl.BlockSpec((2, BN, D), lambda i: (0, i, 0)),
            pl.BlockSpec((BN, 2 * D), lambda i: (i, 0)),
            pl.BlockSpec((2 * D, D), lambda i: (0, 0)),
            pl.BlockSpec((1, D), lambda i: (0, 0)),
            pl.BlockSpec((BN, 1), lambda i: (i, 0)),
        ],
        out_specs=pl.BlockSpec((G, D), lambda i: (0, 0)),
        out_shape=jax.ShapeDtypeStruct((G, D), jnp.float32),
        scratch_shapes=[
            pltpu.VMEM((G, D), jnp.float32),
            pltpu.VMEM((G, 1), jnp.float32),
        ],
    )(p, h, wroot, b, batch2d)


def kernel(x, edge_index, batch, W1_rel, W1_root, b1, W2_rel, W2_root, b2,
           W3_rel, W3_root, b3):
    f32 = jnp.float32
    x_pad = jnp.concatenate([x, jnp.zeros((N_PAD - N, D), f32)], axis=0)
    src = edge_index[0]
    dst = edge_index[1]
    pad_src = jnp.arange(E_PAD - E, dtype=jnp.int32) % N
    src_p = jnp.concatenate([src, pad_src]).reshape(NW, EC, CHUNK)
    pad_dst = N + (jnp.arange(E_PAD - E, dtype=jnp.int32) % (N_PAD - N))
    dst_p = jnp.concatenate([dst, pad_dst]).reshape(NW, EC, CHUNK)
    zeros_hbm = jnp.zeros((RPT, D), f32)
    batch2d = jnp.concatenate(
        [batch, jnp.full((N_PAD - N,), G, jnp.int32)]
    ).astype(f32).reshape(N_PAD, 1)
    b1r = b1.reshape(1, -1)
    b2r = b2.reshape(1, -1)
    b3r = b3.reshape(1, -1)

    p1 = _edge_seg_sum(x_pad, src_p, dst_p, zeros_hbm)
    h1 = _dense_layer(p1, x_pad, W1_rel, W1_root, b1r, D)
    p2 = _edge_seg_sum(h1, src_p, dst_p, zeros_hbm)
    h2, y3 = _dense_layer2(p2, h1, W2_rel, W2_root, b2r, W3_rel)
    p3 = _edge_seg_sum(y3, src_p, dst_p, zeros_hbm)
    return _dense_layer3_pool(p3, h2, W3_root, b3r, batch2d)


# speedup vs baseline: 3.7521x; 1.0131x over previous
import functools

import jax
import jax.numpy as jnp
from jax import lax
from jax.experimental import pallas as pl
from jax.experimental.pallas import tpu as pltpu
from jax.experimental.pallas import tpu_sc as plsc

N = 10000
E = 320000
G = 64
D = 128

NC = 2
NS = 16
NW = NC * NS
CHUNK = 128

EC = 80
E_PAD = NW * EC * CHUNK
N_PAD = 10112
RPT = N_PAD // NS

def _seg_sum_kernel(n_chunks, out_rows):
    mesh = plsc.VectorSubcoreMesh(core_axis_name="c", subcore_axis_name="s")
    rpt = out_rows // NS

    npass = 2
    hp = n_chunks // npass
    assert hp % 2 == 0 and hp % 8 == 0

    @functools.partial(
        pl.kernel,
        mesh=mesh,
        out_type=jax.ShapeDtypeStruct((NC, out_rows, D), jnp.float32),
        scratch_types=[
            pltpu.VMEM((hp, CHUNK), jnp.int32),
            pltpu.VMEM((hp, CHUNK), jnp.int32),
            pltpu.VMEM((CHUNK, D), jnp.float32),
            pltpu.VMEM((CHUNK, D), jnp.float32),
            pltpu.VMEM_SHARED((out_rows, D), jnp.float32),
            pltpu.SemaphoreType.DMA,
            pltpu.SemaphoreType.DMA,
            pltpu.SemaphoreType.DMA,
            pltpu.SemaphoreType.DMA,
        ],
    )
    def k(table, src, dst, zeros, out, src_v, dst_v, b0, b1, agg_sh,
          t0, t1, g0, g1):
        c = lax.axis_index("c")
        s = lax.axis_index("s")
        wid = s * NC + c
        pltpu.sync_copy(zeros.at[pl.ds(0, rpt)], agg_sh.at[pl.ds(s * rpt, rpt)])
        plsc.subcore_barrier()

        for p in range(npass):
            pltpu.sync_copy(src.at[wid, pl.ds(p * hp, hp)], src_v)
            pltpu.sync_copy(dst.at[wid, pl.ds(p * hp, hp)], dst_v)

            pltpu.async_copy(table.at[src_v.at[0]], b0, g0)
            pltpu.async_copy(table.at[src_v.at[1]], b1, g1)

            def body(i, carry):
                j = 2 * i
                pltpu.make_async_copy(table.at[src_v.at[j]], b0, g0).wait()
                cs0 = pltpu.async_copy(b0, agg_sh.at[dst_v.at[j]], t0,
                                       add=True)
                pltpu.make_async_copy(table.at[src_v.at[j + 1]], b1,
                                      g1).wait()
                cs1 = pltpu.async_copy(b1, agg_sh.at[dst_v.at[j + 1]], t1,
                                       add=True)
                cs0.wait()
                pltpu.async_copy(table.at[src_v.at[j + 2]], b0, g0)
                cs1.wait()
                pltpu.async_copy(table.at[src_v.at[j + 3]], b1, g1)
                return carry

            lax.fori_loop(0, hp // 2 - 1, body, 0)

            j = hp - 2
            pltpu.make_async_copy(table.at[src_v.at[j]], b0, g0).wait()
            cs0 = pltpu.async_copy(b0, agg_sh.at[dst_v.at[j]], t0, add=True)
            pltpu.make_async_copy(table.at[src_v.at[j + 1]], b1, g1).wait()
            cs1 = pltpu.async_copy(b1, agg_sh.at[dst_v.at[j + 1]], t1,
                                   add=True)
            cs0.wait()
            cs1.wait()

        plsc.subcore_barrier()
        pltpu.sync_copy(agg_sh.at[pl.ds(s * rpt, rpt)],
                        out.at[c, pl.ds(s * rpt, rpt)])

    return k


_edge_seg_sum = _seg_sum_kernel(EC, N_PAD)


BN = 632
TC_GRID = N_PAD // BN


def _elu(v):
    return jnp.where(v > 0, v, jnp.exp(v) - 1.0)


def _dense_body(p_ref, h_ref, wrel_ref, wroot_ref, b_ref, out_ref):
    agg = p_ref[0] + p_ref[1]
    acc = jnp.dot(agg, wrel_ref[...], preferred_element_type=jnp.float32)
    acc += jnp.dot(h_ref[...], wroot_ref[...], preferred_element_type=jnp.float32)
    out_ref[...] = _elu(acc + b_ref[...])


def _dense_layer(p, h, wrel, wroot, b, d_out):
    d_in = h.shape[-1]
    return pl.pallas_call(
        _dense_body,
        grid=(TC_GRID,),
        in_specs=[
            pl.BlockSpec((2, BN, D), lambda i: (0, i, 0)),
            pl.BlockSpec((BN, d_in), lambda i: (i, 0)),
            pl.BlockSpec((D, d_out), lambda i: (0, 0)),
            pl.BlockSpec((d_in, d_out), lambda i: (0, 0)),
            pl.BlockSpec((1, d_out), lambda i: (0, 0)),
        ],
        out_specs=pl.BlockSpec((BN, d_out), lambda i: (i, 0)),
        out_shape=jax.ShapeDtypeStruct((N_PAD, d_out), jnp.float32),
    )(p, h, wrel, wroot, b)


def _dense2_body(p_ref, h_ref, wrel_ref, wroot_ref, b_ref, w3rel_ref,
                 h2_ref, y3_ref):
    agg = p_ref[0] + p_ref[1]
    acc = jnp.dot(agg, wrel_ref[...], preferred_element_type=jnp.float32)
    acc += jnp.dot(h_ref[...], wroot_ref[...], preferred_element_type=jnp.float32)
    h2 = _elu(acc + b_ref[...])
    h2_ref[...] = h2
    y3_ref[...] = jnp.dot(h2, w3rel_ref[...], preferred_element_type=jnp.float32)


def _dense_layer2(p, h, wrel, wroot, b, w3rel):
    return pl.pallas_call(
        _dense2_body,
        grid=(TC_GRID,),
        in_specs=[
            pl.BlockSpec((2, BN, D), lambda i: (0, i, 0)),
            pl.BlockSpec((BN, D), lambda i: (i, 0)),
            pl.BlockSpec((D, 2 * D), lambda i: (0, 0)),
            pl.BlockSpec((D, 2 * D), lambda i: (0, 0)),
            pl.BlockSpec((1, 2 * D), lambda i: (0, 0)),
            pl.BlockSpec((2 * D, D), lambda i: (0, 0)),
        ],
        out_specs=[
            pl.BlockSpec((BN, 2 * D), lambda i: (i, 0)),
            pl.BlockSpec((BN, D), lambda i: (i, 0)),
        ],
        out_shape=[
            jax.ShapeDtypeStruct((N_PAD, 2 * D), jnp.float32),
            jax.ShapeDtypeStruct((N_PAD, D), jnp.float32),
        ],
    )(p, h, wrel, wroot, b, w3rel)


def _dense3_body(p_ref, h_ref, wroot_ref, b_ref, batch_ref, out_ref,
                 acc_ref, cnt_ref):
    i = pl.program_id(0)
    agg = p_ref[0] + p_ref[1]
    acc = agg + jnp.dot(h_ref[...], wroot_ref[...],
                        preferred_element_type=jnp.float32)
    h3 = _elu(acc + b_ref[...])
    gids = jax.lax.broadcasted_iota(jnp.int32, (BN, G), 1).astype(jnp.float32)
    m = (batch_ref[...] == gids).astype(jnp.float32)
    psum = jax.lax.dot_general(m, h3, (((0,), (0,)), ((), ())),
                               preferred_element_type=jnp.float32)
    pcnt = jnp.sum(m, axis=0)[:, None]

    @pl.when(i == 0)
    def _():
        acc_ref[...] = psum
        cnt_ref[...] = pcnt

    @pl.when(i > 0)
    def _():
        acc_ref[...] += psum
        cnt_ref[...] += pcnt

    @pl.when(i == TC_GRID - 1)
    def _():
        out_ref[...] = acc_ref[...] / jnp.maximum(cnt_ref[...], 1.0)


def _dense_layer3_pool(p, h, wroot, b, batch2d):
    return pl.pallas_call(
        _dense3_body,
        grid=(TC_GRID,),
        in_specs=[
            pl.BlockSpec((2, BN, D), lambda i: (0, i, 0)),
            pl.BlockSpec((BN, 2 * D), lambda i: (i, 0)),
            pl.BlockSpec((2 * D, D), lambda i: (0, 0)),
            pl.BlockSpec((1, D), lambda i: (0, 0)),
            pl.BlockSpec((BN, 1), lambda i: (i, 0)),
        ],
        out_specs=pl.BlockSpec((G, D), lambda i: (0, 0)),
        out_shape=jax.ShapeDtypeStruct((G, D), jnp.float32),
        scratch_shapes=[
            pltpu.VMEM((G, D), jnp.float32),
            pltpu.VMEM((G, 1), jnp.float32),
        ],
    )(p, h, wroot, b, batch2d)


def kernel(x, edge_index, batch, W1_rel, W1_root, b1, W2_rel, W2_root, b2,
           W3_rel, W3_root, b3):
    f32 = jnp.float32
    x_pad = jnp.concatenate([x, jnp.zeros((N_PAD - N, D), f32)], axis=0)
    src = edge_index[0]
    dst = edge_index[1]
    pad_src = jnp.arange(E_PAD - E, dtype=jnp.int32) % N
    src_p = jnp.concatenate([src, pad_src]).reshape(NW, EC, CHUNK)
    pad_dst = N + (jnp.arange(E_PAD - E, dtype=jnp.int32) % (N_PAD - N))
    dst_p = jnp.concatenate([dst, pad_dst]).reshape(NW, EC, CHUNK)
    zeros_hbm = jnp.zeros((RPT, D), f32)
    batch2d = jnp.concatenate(
        [batch, jnp.full((N_PAD - N,), G, jnp.int32)]
    ).astype(f32).reshape(N_PAD, 1)
    b1r = b1.reshape(1, -1)
    b2r = b2.reshape(1, -1)
    b3r = b3.reshape(1, -1)

    p1 = _edge_seg_sum(x_pad, src_p, dst_p, zeros_hbm)
    h1 = _dense_layer(p1, x_pad, W1_rel, W1_root, b1r, D)
    p2 = _edge_seg_sum(h1, src_p, dst_p, zeros_hbm)
    h2, y3 = _dense_layer2(p2, h1, W2_rel, W2_root, b2r, W3_rel)
    p3 = _edge_seg_sum(y3, src_p, dst_p, zeros_hbm)
    return _dense_layer3_pool(p3, h2, W3_root, b3r, batch2d)
